# baseline (speedup 1.0000x reference)
import jax
import jax.numpy as jnp
from jax.experimental import pallas as pl
from jax.experimental.pallas import tpu as pltpu


def _probe(x_hbm, out_ref, b0, b1, b2, b3, sem_ref):
    bufs = [b0, b1, b2, b3]
    sizes = [8, 8, 8, 4]
    cps = [pltpu.make_async_copy(x_hbm.at[:, pl.ds(8 * t, sizes[t]), :],
                                 bufs[t], sem_ref.at[t]) for t in range(4)]
    for c in cps:
        c.start()
    for c in cps:
        c.wait()
    out_ref[...] = jnp.zeros_like(out_ref) + b0[0, 0, 0]


def kernel(x, train, W1, b1, Wk, bk, Wq, bq, Wv, bv, Wskip, conv_bias,
           bn_gamma, bn_beta, fc_W, fc_b):
    B = x.shape[0]
    return pl.pallas_call(
        _probe,
        in_specs=[pl.BlockSpec(memory_space=pl.ANY)],
        out_specs=pl.BlockSpec(memory_space=pltpu.MemorySpace.VMEM),
        out_shape=jax.ShapeDtypeStruct((B, 10), jnp.float32),
        scratch_shapes=[pltpu.VMEM((B, 8, 28), jnp.float32)] * 3 + [
            pltpu.VMEM((B, 4, 28), jnp.float32),
            pltpu.SemaphoreType.DMA((4,)),
        ],
    )(x)


# speedup vs baseline: 1.4321x; 1.4321x over previous
import jax
import jax.numpy as jnp
import numpy as np
from jax.experimental import pallas as pl
from jax.experimental.pallas import tpu as pltpu

_H = 16


def _selectors(n):
    ri = jax.lax.broadcasted_iota(jnp.int32, (2 * n, n), 0)
    ci = jax.lax.broadcasted_iota(jnp.int32, (2 * n, n), 1)
    se = (ri == 2 * ci).astype(jnp.float32)
    so = (ri == 2 * ci + 1).astype(jnp.float32)
    return se, so


def _fused(x1_ref, w1_ref, b1_ref, wk_ref, bk_ref, wq_ref, bq_ref,
           wv_ref, bv_ref, wskip_ref, cb_ref, gamma_ref, beta_ref,
           fcw_ref, fcb_ref, out_ref):
    f32 = jnp.float32
    bf16 = jnp.bfloat16
    dot = lambda a, b: jnp.dot(a, b, preferred_element_type=f32)
    se32, so32 = _selectors(_H)
    se16, so16 = _selectors(_H // 2)

    xab = dot(x1_ref[...].astype(bf16), w1_ref[...].astype(bf16)) + b1_ref[...]
    x2 = jnp.maximum(jnp.maximum(dot(xab, se32), dot(xab, so32)), 0.0)

    k = dot(x2, wk_ref[...]) + bk_ref[...]
    q = dot(x2, wq_ref[...]) + bq_ref[...]
    v = dot(x2, wv_ref[...]) + bv_ref[...]
    b = x2.shape[0]
    gate = 0.5 + 0.5 * jnp.tanh(0.5 * (k + q))
    agg = float(b) * gate * v
    x3 = agg + dot(x2, wskip_ref[...]) + cb_ref[...]
    x3 = x3 * (gamma_ref[...] * (1.0 / np.sqrt(1.0 + 1e-5))) + beta_ref[...]

    x4 = jnp.maximum(dot(x3, se16), dot(x3, so16))
    x4e = jnp.concatenate([x4, jnp.ones((b, 1), f32)], axis=1)

    x4b = x4.astype(bf16)
    x4hb = (x4 * 0.5).astype(bf16)
    x4eb = x4e.astype(bf16)
    colsum = jnp.sum(x4e, axis=0, keepdims=True)
    nj = b // 128
    parts = []
    for j in range(nj):
        gj = jax.lax.dot_general(x4hb[j * 128:(j + 1) * 128, :], x4b,
                                 (((1,), (1,)), ((), ())),
                                 preferred_element_type=f32)
        tj = jnp.tanh(gj.astype(bf16))
        parts.append(dot(tj, x4eb))
    rr = 0.5 * (jnp.concatenate(parts, axis=0) + colsum)
    hh = _H // 2
    x6 = rr[:, :hh] / rr[:, hh:hh + 1] + x4
    out_ref[...] = dot(x6, fcw_ref[...]) + fcb_ref[...]


def kernel(x, train, W1, b1, Wk, bk, Wq, bq, Wv, bv, Wskip, conv_bias,
           bn_gamma, bn_beta, fc_W, fc_b):
    B = x.shape[0]
    d = x.shape[1] * x.shape[2]
    x1 = x.reshape(B, d)
    row = lambda t: t.reshape(1, t.shape[0])

    out = pl.pallas_call(
        _fused,
        out_shape=jax.ShapeDtypeStruct((B, fc_W.shape[1]), jnp.float32),
    )(x1, W1, row(b1), Wk, row(bk), Wq, row(bq), Wv, row(bv), Wskip,
      row(conv_bias), row(bn_gamma), row(bn_beta), fc_W, row(fc_b))
    return out


# speedup vs baseline: 1.4622x; 1.0210x over previous
import jax
import jax.numpy as jnp
import numpy as np
from jax.experimental import pallas as pl
from jax.experimental.pallas import tpu as pltpu

_H = 16


def _selectors(n):
    ri = jax.lax.broadcasted_iota(jnp.int32, (2 * n, n), 0)
    ci = jax.lax.broadcasted_iota(jnp.int32, (2 * n, n), 1)
    se = (ri == 2 * ci).astype(jnp.float32)
    so = (ri == 2 * ci + 1).astype(jnp.float32)
    return se, so


def _fused(x1_ref, w1_ref, b1_ref, wk_ref, bk_ref, wq_ref, bq_ref,
           wv_ref, bv_ref, wskip_ref, cb_ref, gamma_ref, beta_ref,
           fcw_ref, fcb_ref, out_ref):
    f32 = jnp.float32
    bf16 = jnp.bfloat16
    dot = lambda a, b: jnp.dot(a, b, preferred_element_type=f32)
    se32, so32 = _selectors(_H)
    se16, so16 = _selectors(_H // 2)

    xab = dot(x1_ref[...].astype(bf16), w1_ref[...].astype(bf16)) + b1_ref[...]
    x2 = jnp.maximum(jnp.maximum(dot(xab, se32), dot(xab, so32)), 0.0)

    k = dot(x2, wk_ref[...]) + bk_ref[...]
    q = dot(x2, wq_ref[...]) + bq_ref[...]
    v = dot(x2, wv_ref[...]) + bv_ref[...]
    b = x2.shape[0]
    gate = 0.5 + 0.5 * jnp.tanh(0.5 * (k + q))
    agg = float(b) * gate * v
    x3 = agg + dot(x2, wskip_ref[...]) + cb_ref[...]
    x3 = x3 * (gamma_ref[...] * (1.0 / np.sqrt(1.0 + 1e-5))) + beta_ref[...]

    x4 = jnp.maximum(dot(x3, se16), dot(x3, so16))
    x4e = jnp.concatenate([x4, jnp.ones((b, 1), f32)], axis=1)

    x4b = x4.astype(bf16)
    x4hb = (x4 * 0.5).astype(bf16)
    g = jax.lax.dot_general(x4hb, x4b, (((1,), (1,)), ((), ())),
                            preferred_element_type=f32)
    t = jnp.tanh(g.astype(bf16))
    colsum = jnp.sum(x4e, axis=0, keepdims=True)
    rr = 0.5 * (dot(t, x4e.astype(bf16)) + colsum)
    hh = _H // 2
    x6 = rr[:, :hh] / rr[:, hh:hh + 1] + x4
    out_ref[...] = dot(x6, fcw_ref[...]) + fcb_ref[...]


def kernel(x, train, W1, b1, Wk, bk, Wq, bq, Wv, bv, Wskip, conv_bias,
           bn_gamma, bn_beta, fc_W, fc_b):
    B = x.shape[0]
    d = x.shape[1] * x.shape[2]
    x1 = x.reshape(B, d)
    row = lambda t: t.reshape(1, t.shape[0])

    out = pl.pallas_call(
        _fused,
        out_shape=jax.ShapeDtypeStruct((B, fc_W.shape[1]), jnp.float32),
    )(x1, W1, row(b1), Wk, row(bk), Wq, row(bq), Wv, row(bv), Wskip,
      row(conv_bias), row(bn_gamma), row(bn_beta), fc_W, row(fc_b))
    return out


# speedup vs baseline: 1.5829x; 1.0825x over previous
import jax
import jax.numpy as jnp
import numpy as np
from jax.experimental import pallas as pl
from jax.experimental.pallas import tpu as pltpu

_H = 16


def _selectors(n):
    ri = jax.lax.broadcasted_iota(jnp.int32, (2 * n, n), 0)
    ci = jax.lax.broadcasted_iota(jnp.int32, (2 * n, n), 1)
    se = (ri == 2 * ci).astype(jnp.float32)
    so = (ri == 2 * ci + 1).astype(jnp.float32)
    return se, so


def _fused(x1_ref, w1_ref, b1_ref, wk_ref, bk_ref, wq_ref, bq_ref,
           wv_ref, bv_ref, wskip_ref, cb_ref, gamma_ref, beta_ref,
           fcw_ref, fcb_ref, out_ref):
    f32 = jnp.float32
    bf16 = jnp.bfloat16
    dot = lambda a, b: jnp.dot(a, b, preferred_element_type=f32)
    se32, so32 = _selectors(_H)
    se16, so16 = _selectors(_H // 2)

    xab = dot(x1_ref[...], w1_ref[...].astype(bf16)) + b1_ref[...]
    x2 = jnp.maximum(jnp.maximum(dot(xab, se32), dot(xab, so32)), 0.0)

    k = dot(x2, wk_ref[...]) + bk_ref[...]
    q = dot(x2, wq_ref[...]) + bq_ref[...]
    v = dot(x2, wv_ref[...]) + bv_ref[...]
    b = x2.shape[0]
    gate = 0.5 + 0.5 * jnp.tanh(0.5 * (k + q))
    agg = float(b) * gate * v
    x3 = agg + dot(x2, wskip_ref[...]) + cb_ref[...]
    x3 = x3 * (gamma_ref[...] * (1.0 / np.sqrt(1.0 + 1e-5))) + beta_ref[...]

    x4 = jnp.maximum(dot(x3, se16), dot(x3, so16))
    x4e = jnp.concatenate([x4, jnp.ones((b, 1), f32)], axis=1)

    x4b = x4.astype(bf16)
    x4hb = (x4 * 0.5).astype(bf16)
    g = jax.lax.dot_general(x4hb, x4b, (((1,), (1,)), ((), ())),
                            preferred_element_type=f32)
    t = jnp.tanh(g.astype(bf16))
    colsum = jnp.sum(x4e, axis=0, keepdims=True)
    rr = 0.5 * (dot(t, x4e.astype(bf16)) + colsum)
    hh = _H // 2
    x6 = rr[:, :hh] / rr[:, hh:hh + 1] + x4
    out_ref[...] = dot(x6, fcw_ref[...]) + fcb_ref[...]


def kernel(x, train, W1, b1, Wk, bk, Wq, bq, Wv, bv, Wskip, conv_bias,
           bn_gamma, bn_beta, fc_W, fc_b):
    B = x.shape[0]
    d = x.shape[1] * x.shape[2]
    x1 = x.reshape(B, d).astype(jnp.bfloat16)
    row = lambda t: t.reshape(1, t.shape[0])

    out = pl.pallas_call(
        _fused,
        out_shape=jax.ShapeDtypeStruct((B, fc_W.shape[1]), jnp.float32),
    )(x1, W1, row(b1), Wk, row(bk), Wq, row(bq), Wv, row(bv), Wskip,
      row(conv_bias), row(bn_gamma), row(bn_beta), fc_W, row(fc_b))
    return out


# speedup vs baseline: 1.6141x; 1.0197x over previous
import jax
import jax.numpy as jnp
import numpy as np
from jax.experimental import pallas as pl
from jax.experimental.pallas import tpu as pltpu

_H = 16


def _selectors(n):
    ri = jax.lax.broadcasted_iota(jnp.int32, (2 * n, n), 0)
    ci = jax.lax.broadcasted_iota(jnp.int32, (2 * n, n), 1)
    se = (ri == 2 * ci).astype(jnp.float32)
    so = (ri == 2 * ci + 1).astype(jnp.float32)
    return se, so


def _fused(x1_ref, w1_ref, b1_ref, wk_ref, bk_ref, wq_ref, bq_ref,
           wv_ref, bv_ref, wskip_ref, cb_ref, gamma_ref, beta_ref,
           fcw_ref, fcb_ref, out_ref):
    f32 = jnp.float32
    bf16 = jnp.bfloat16
    dot = lambda a, b: jnp.dot(a, b, preferred_element_type=f32)
    se32, so32 = _selectors(_H)
    se16, so16 = _selectors(_H // 2)

    xab = dot(x1_ref[...], w1_ref[...].astype(bf16)) + b1_ref[...]
    x2 = jnp.maximum(jnp.maximum(dot(xab, se32), dot(xab, so32)), 0.0)

    b = x2.shape[0]
    gam = gamma_ref[...] * (1.0 / np.sqrt(1.0 + 1e-5))
    kq = dot(x2, wk_ref[...] + wq_ref[...]) + (bk_ref[...] + bq_ref[...])
    gate = 0.5 + 0.5 * jnp.tanh(0.5 * kq)
    vs = dot(x2, wv_ref[...] * (float(b) * gam)) + bv_ref[...] * (float(b) * gam)
    skip = dot(x2, wskip_ref[...] * gam)
    x3 = gate * vs + skip + (cb_ref[...] * gam + beta_ref[...])

    x4 = jnp.maximum(dot(x3, se16), dot(x3, so16))
    x4e = jnp.concatenate([x4, jnp.ones((b, 1), f32)], axis=1)

    x4b = x4.astype(bf16)
    x4hb = (x4 * 0.5).astype(bf16)
    g = jax.lax.dot_general(x4hb, x4b, (((1,), (1,)), ((), ())),
                            preferred_element_type=f32)
    t = jnp.tanh(g.astype(bf16))
    colsum = jnp.sum(x4e, axis=0, keepdims=True)
    rr = 0.5 * (dot(t, x4e.astype(bf16)) + colsum)
    hh = _H // 2
    x6 = rr[:, :hh] / rr[:, hh:hh + 1] + x4
    out_ref[...] = dot(x6, fcw_ref[...]) + fcb_ref[...]


def kernel(x, train, W1, b1, Wk, bk, Wq, bq, Wv, bv, Wskip, conv_bias,
           bn_gamma, bn_beta, fc_W, fc_b):
    B = x.shape[0]
    d = x.shape[1] * x.shape[2]
    x1 = x.reshape(B, d).astype(jnp.bfloat16)
    row = lambda t: t.reshape(1, t.shape[0])

    out = pl.pallas_call(
        _fused,
        out_shape=jax.ShapeDtypeStruct((B, fc_W.shape[1]), jnp.float32),
    )(x1, W1, row(b1), Wk, row(bk), Wq, row(bq), Wv, row(bv), Wskip,
      row(conv_bias), row(bn_gamma), row(bn_beta), fc_W, row(fc_b))
    return out
